# baseline (speedup 1.0000x reference)
import functools

import jax
import jax.numpy as jnp
from jax import lax
from jax.experimental import pallas as pl
from jax.experimental.pallas import tpu as pltpu
from jax.experimental.pallas import tpu_sc as plsc

N = 10000
E = 320000
F = 128
K = 8
HID = 512
OUT = 128
SLOPE = 0.2

NC = 2
NS = 16
NW = NC * NS
CHUNK = 128
M = E + N
NBUF = 2
GB = 8
CPT = -(-M // (NW * CHUNK * GB)) * GB
CAPT = CPT * CHUNK
MPAD = CAPT * NW
ACC = 10112
RPT = ACC // NS
DEGW = 16


def _make_sc_mesh():
    return plsc.VectorSubcoreMesh(core_axis_name="c", subcore_axis_name="s")


def _sc_degree(degslab, zeros16, ones16):
    mesh = _make_sc_mesh()

    def body(degslab_hbm, zeros16_hbm, ones16_hbm, out_hbm, degv, onesv, acc, sem):
        c = lax.axis_index("c")
        s = lax.axis_index("s")
        t = c * NS + s
        pltpu.sync_copy(degslab_hbm.at[t], degv)
        pltpu.sync_copy(ones16_hbm, onesv)
        pltpu.sync_copy(zeros16_hbm.at[pl.ds(s * RPT, RPT)],
                        acc.at[pl.ds(s * RPT, RPT)])
        plsc.subcore_barrier()

        def chunk(j, carry):
            pltpu.sync_copy(onesv, acc.at[degv.at[j]], add=True)
            return carry

        lax.fori_loop(0, CPT, chunk, 0)
        plsc.subcore_barrier()
        pltpu.sync_copy(acc.at[pl.ds(s * RPT, RPT)],
                        out_hbm.at[c, pl.ds(s * RPT, RPT)])

    f = pl.kernel(
        body,
        out_type=jax.ShapeDtypeStruct((NC, ACC, DEGW), jnp.float32),
        mesh=mesh,
        scratch_types=[
            pltpu.VMEM((CPT, CHUNK), jnp.int32),
            pltpu.VMEM((CHUNK, DEGW), jnp.float32),
            pltpu.VMEM_SHARED((ACC, DEGW), jnp.float32),
            pltpu.SemaphoreType.DMA,
        ],
    )
    return f(degslab, zeros16, ones16)


def _sc_prop(g, rowslab, colslab, zeros):
    mesh = _make_sc_mesh()

    def body(g_hbm, rowslab_hbm, colslab_hbm, zeros_hbm, out_hbm,
             rowv, coli, bufs, acc, gsems):
        c = lax.axis_index("c")
        s = lax.axis_index("s")
        t = c * NS + s
        pltpu.sync_copy(rowslab_hbm.at[t], rowv)
        pltpu.sync_copy(zeros_hbm.at[pl.ds(s * RPT, RPT)],
                        acc.at[pl.ds(s * RPT, RPT)])

        def g_start(j, u2):
            pltpu.async_copy(g_hbm.at[rowv.at[j]], bufs.at[u2], gsems[u2])

        def g_wait(j, u2):
            pltpu.make_async_copy(g_hbm.at[rowv.at[j]], bufs.at[u2],
                                  gsems[u2]).wait()

        def s_sync(u2, u):
            pltpu.sync_copy(bufs.at[u2], acc.at[coli.at[u]], add=True)

        g_start(0, 0)
        plsc.subcore_barrier()

        def group(grp, carry):
            pltpu.sync_copy(colslab_hbm.at[t, pl.ds(grp * GB, GB)], coli)
            base = grp * GB
            for u in range(GB):
                j = base + u
                g_wait(j, u % 2)
                g_start(j + 1, (u + 1) % 2)
                s_sync(u % 2, u)
            return carry

        lax.fori_loop(0, CPT // GB - 1, group, 0)

        base = CPT - GB
        pltpu.sync_copy(colslab_hbm.at[t, pl.ds(base, GB)], coli)
        for u in range(GB):
            j = base + u
            g_wait(j, u % 2)
            if u < GB - 1:
                g_start(j + 1, (u + 1) % 2)
            s_sync(u % 2, u)

        plsc.subcore_barrier()
        pltpu.sync_copy(acc.at[pl.ds(s * RPT, RPT)],
                        out_hbm.at[c, pl.ds(s * RPT, RPT)])

    f = pl.kernel(
        body,
        out_type=jax.ShapeDtypeStruct((NC, ACC, F), jnp.float32),
        mesh=mesh,
        scratch_types=[
            pltpu.VMEM((CPT, CHUNK), jnp.int32),
            pltpu.VMEM((GB, CHUNK), jnp.int32),
            pltpu.VMEM((NBUF, CHUNK, F), jnp.float32),
            pltpu.VMEM_SHARED((ACC, F), jnp.float32),
            [pltpu.SemaphoreType.DMA] * NBUF,
        ],
    )
    return f(g, rowslab, colslab, zeros)


def _prep_kernel(degp_ref, feat_ref, g0_ref, invdeg_ref, rsq_ref):
    deg = degp_ref[0, :N, 0] + degp_ref[1, :N, 0]
    rsq = lax.rsqrt(deg)
    pad = jnp.zeros((ACC - N,), jnp.float32)
    invdeg_ref[...] = jnp.concatenate([rsq * rsq, pad])
    rsq_ref[...] = jnp.concatenate([rsq, pad])
    g0_ref[...] = feat_ref[...] * rsq[:, None]


def _tc_prep(degp, feature):
    return pl.pallas_call(
        _prep_kernel,
        out_shape=(
            jax.ShapeDtypeStruct((N, F), jnp.float32),
            jax.ShapeDtypeStruct((ACC,), jnp.float32),
            jax.ShapeDtypeStruct((ACC,), jnp.float32),
        ),
    )(degp, feature)


def _combine_kernel(p_ref, invdeg_ref, rsq_ref, g_ref, h_ref):
    acc = p_ref[0, :N, :] + p_ref[1, :N, :]
    g_ref[...] = acc * invdeg_ref[:N][:, None]
    h_ref[...] = acc * rsq_ref[:N][:, None]


def _tc_combine(p, invdeg, rsq):
    return pl.pallas_call(
        _combine_kernel,
        out_shape=(
            jax.ShapeDtypeStruct((N, F), jnp.float32),
            jax.ShapeDtypeStruct((N, F), jnp.float32),
        ),
    )(p, invdeg, rsq)


BN = 1000
NB = N // BN


def _enc1_kernel(b1_ref, w1_ref, *refs):
    x_refs = refs[:K + 1]
    h1_ref, ps_ref, psq_ref = refs[K + 1:]
    acc = jnp.broadcast_to(b1_ref[...], (BN, HID)).astype(jnp.float32)
    for k in range(K + 1):
        acc = acc + jnp.dot(x_refs[k][...], w1_ref[k],
                            preferred_element_type=jnp.float32)
    h1 = jnp.where(acc >= 0, acc, SLOPE * acc)
    h1_ref[...] = h1
    ps_ref[...] = jnp.pad(jnp.sum(h1, axis=0, keepdims=True), ((0, 7), (0, 0)))
    psq_ref[...] = jnp.pad(jnp.sum(h1 * h1, axis=0, keepdims=True), ((0, 7), (0, 0)))


def _tc_enc1(xs, w1r, b1):
    x_specs = [pl.BlockSpec((BN, F), lambda i: (i, 0)) for _ in range(K + 1)]
    return pl.pallas_call(
        _enc1_kernel,
        grid=(NB,),
        in_specs=[
            pl.BlockSpec((HID,), lambda i: (0,)),
            pl.BlockSpec((K + 1, F, HID), lambda i: (0, 0, 0)),
        ] + x_specs,
        out_specs=(
            pl.BlockSpec((BN, HID), lambda i: (i, 0)),
            pl.BlockSpec((8, HID), lambda i: (i, 0)),
            pl.BlockSpec((8, HID), lambda i: (i, 0)),
        ),
        out_shape=(
            jax.ShapeDtypeStruct((N, HID), jnp.float32),
            jax.ShapeDtypeStruct((NB * 8, HID), jnp.float32),
            jax.ShapeDtypeStruct((NB * 8, HID), jnp.float32),
        ),
    )(b1, w1r, *xs)


def _enc2_kernel(h1_ref, ps_ref, psq_ref, gamma_ref, beta_ref, w2_ref, b2_ref,
                 wd_ref, bd_ref, zn_ref, r_ref):
    mu = jnp.sum(ps_ref[...], axis=0) / N
    ex2 = jnp.sum(psq_ref[...], axis=0) / N
    var = ex2 - mu * mu
    rstd = lax.rsqrt(var + 1e-5)
    h1 = (h1_ref[...] - mu) * (rstd * gamma_ref[...]) + beta_ref[...]
    z = jnp.dot(h1, w2_ref[...], preferred_element_type=jnp.float32) + b2_ref[...]
    r_ref[...] = jnp.dot(z, wd_ref[...], preferred_element_type=jnp.float32) + bd_ref[...]
    nrm = jnp.sqrt(jnp.sum(z * z, axis=1, keepdims=True))
    zn_ref[...] = z / jnp.maximum(nrm, 1e-12)


def _tc_enc2(h1, ps, psq, gamma, beta, W2, b2, Wd, bd):
    return pl.pallas_call(
        _enc2_kernel,
        grid=(NB,),
        in_specs=[
            pl.BlockSpec((BN, HID), lambda i: (i, 0)),
            pl.BlockSpec((NB * 8, HID), lambda i: (0, 0)),
            pl.BlockSpec((NB * 8, HID), lambda i: (0, 0)),
            pl.BlockSpec((HID,), lambda i: (0,)),
            pl.BlockSpec((HID,), lambda i: (0,)),
            pl.BlockSpec((HID, OUT), lambda i: (0, 0)),
            pl.BlockSpec((OUT,), lambda i: (0,)),
            pl.BlockSpec((OUT, F), lambda i: (0, 0)),
            pl.BlockSpec((F,), lambda i: (0,)),
        ],
        out_specs=(
            pl.BlockSpec((BN, OUT), lambda i: (i, 0)),
            pl.BlockSpec((BN, F), lambda i: (i, 0)),
        ),
        out_shape=(
            jax.ShapeDtypeStruct((N, OUT), jnp.float32),
            jax.ShapeDtypeStruct((N, F), jnp.float32),
        ),
    )(h1, ps, psq, gamma, beta, W2, b2, Wd, bd)


def kernel(feature, edge_index, W1, b1, gamma, beta, W2, b2, Wd, bd):
    row = edge_index[0]
    col = edge_index[1]
    self_mask = row == col
    loop_idx = jnp.arange(N, dtype=jnp.int32)
    trash = N + jnp.mod(row, ACC - N)

    prop_row = jnp.concatenate([row, loop_idx])
    prop_col = jnp.concatenate([jnp.where(self_mask, trash, col), loop_idx])
    deg_row = jnp.concatenate([jnp.where(self_mask, trash, row), loop_idx])

    pad = MPAD - M
    pad_trash = N + jnp.mod(jnp.arange(pad, dtype=jnp.int32), ACC - N)
    pad_zero = jnp.zeros((pad,), jnp.int32)
    prop_row = jnp.concatenate([prop_row, pad_zero]).reshape(NW, CPT, CHUNK)
    prop_col = jnp.concatenate([prop_col, pad_trash]).reshape(NW, CPT, CHUNK)
    deg_row = jnp.concatenate([deg_row, pad_trash]).reshape(NW, CPT, CHUNK)

    zeros = jnp.zeros((ACC, F), jnp.float32)
    zeros16 = jnp.zeros((ACC, DEGW), jnp.float32)
    ones16 = jnp.ones((CHUNK, DEGW), jnp.float32)

    degp = _sc_degree(deg_row, zeros16, ones16)
    g0, invdeg, rsq = _tc_prep(degp, feature)

    xs = [feature]
    g = g0
    for _ in range(K):
        p = _sc_prop(g, prop_row, prop_col, zeros)
        g, h = _tc_combine(p, invdeg, rsq)
        xs.append(h)

    w1r = W1.reshape(K + 1, F, HID)
    h1, ps, psq = _tc_enc1(xs, w1r, b1)
    zn, r = _tc_enc2(h1, ps, psq, gamma, beta, W2, b2, Wd, bd)
    return (zn, r)


# speedup vs baseline: 8.5919x; 8.5919x over previous
import functools

import jax
import jax.numpy as jnp
from jax import lax
from jax.experimental import pallas as pl
from jax.experimental.pallas import tpu as pltpu
from jax.experimental.pallas import tpu_sc as plsc

N = 10000
E = 320000
F = 128
K = 8
HID = 512
OUT = 128
SLOPE = 0.2

NC = 2
NS = 16
NW = NC * NS
CHUNK = 128
M = E + N
NBUF = 2
GB = 8
CPT = -(-M // (NW * CHUNK * GB)) * GB
CAPT = CPT * CHUNK
MPAD = CAPT * NW
ACC = 10112
RPT = ACC // NS
DEGW = 16


def _make_sc_mesh():
    return plsc.VectorSubcoreMesh(core_axis_name="c", subcore_axis_name="s")


def _sc_degree(degslab, zeros16, ones16):
    mesh = _make_sc_mesh()

    def body(degslab_hbm, zeros16_hbm, ones16_hbm, out_hbm, degv, onesv, acc, sem):
        c = lax.axis_index("c")
        s = lax.axis_index("s")
        t = c * NS + s
        pltpu.sync_copy(degslab_hbm.at[t], degv)
        pltpu.sync_copy(ones16_hbm, onesv)
        pltpu.sync_copy(zeros16_hbm.at[pl.ds(s * RPT, RPT)],
                        acc.at[pl.ds(s * RPT, RPT)])
        plsc.subcore_barrier()

        def chunk(j, carry):
            pltpu.sync_copy(onesv, acc.at[degv.at[j]], add=True)
            return carry

        lax.fori_loop(0, CPT, chunk, 0)
        plsc.subcore_barrier()
        pltpu.sync_copy(acc.at[pl.ds(s * RPT, RPT)],
                        out_hbm.at[c, pl.ds(s * RPT, RPT)])

    f = pl.kernel(
        body,
        out_type=jax.ShapeDtypeStruct((NC, ACC, DEGW), jnp.float32),
        mesh=mesh,
        scratch_types=[
            pltpu.VMEM((CPT, CHUNK), jnp.int32),
            pltpu.VMEM((CHUNK, DEGW), jnp.float32),
            pltpu.VMEM_SHARED((ACC, DEGW), jnp.float32),
            pltpu.SemaphoreType.DMA,
        ],
    )
    return f(degslab, zeros16, ones16)


def _sc_prop(g, rowslab, colslab, zeros):
    mesh = _make_sc_mesh()

    def body(g_hbm, rowslab_hbm, colslab_hbm, zeros_hbm, out_hbm,
             rowv, coli, bufs, acc, gsems):
        c = lax.axis_index("c")
        s = lax.axis_index("s")
        t = c * NS + s
        pltpu.sync_copy(rowslab_hbm.at[t], rowv)
        pltpu.sync_copy(zeros_hbm.at[pl.ds(s * RPT, RPT)],
                        acc.at[pl.ds(s * RPT, RPT)])

        def g_start(j, u2):
            pltpu.async_copy(g_hbm.at[rowv.at[j]], bufs.at[u2], gsems[u2])

        def g_wait(j, u2):
            pltpu.make_async_copy(g_hbm.at[rowv.at[j]], bufs.at[u2],
                                  gsems[u2]).wait()

        def s_sync(u2, u):
            pltpu.sync_copy(bufs.at[u2], acc.at[coli.at[u]], add=True)

        g_start(0, 0)
        plsc.subcore_barrier()

        def group(grp, carry):
            pltpu.sync_copy(colslab_hbm.at[t, pl.ds(grp * GB, GB)], coli)
            base = grp * GB
            for u in range(GB):
                j = base + u
                g_wait(j, u % 2)
                g_start(j + 1, (u + 1) % 2)
                s_sync(u % 2, u)
            return carry

        lax.fori_loop(0, CPT // GB - 1, group, 0)

        base = CPT - GB
        pltpu.sync_copy(colslab_hbm.at[t, pl.ds(base, GB)], coli)
        for u in range(GB):
            j = base + u
            g_wait(j, u % 2)
            if u < GB - 1:
                g_start(j + 1, (u + 1) % 2)
            s_sync(u % 2, u)

        plsc.subcore_barrier()
        pltpu.sync_copy(acc.at[pl.ds(s * RPT, RPT)],
                        out_hbm.at[c, pl.ds(s * RPT, RPT)])

    f = pl.kernel(
        body,
        out_type=jax.ShapeDtypeStruct((NC, ACC, F), jnp.float32),
        mesh=mesh,
        scratch_types=[
            pltpu.VMEM((CPT, CHUNK), jnp.int32),
            pltpu.VMEM((GB, CHUNK), jnp.int32),
            pltpu.VMEM((NBUF, CHUNK, F), jnp.float32),
            pltpu.VMEM_SHARED((ACC, F), jnp.float32),
            [pltpu.SemaphoreType.DMA] * NBUF,
        ],
    )
    return f(g, rowslab, colslab, zeros)


def _prep_kernel(degp_ref, feat_ref, g0_ref, invdeg_ref, rsq_ref):
    deg = degp_ref[0, :N, 0] + degp_ref[1, :N, 0]
    rsq = lax.rsqrt(deg)
    pad = jnp.zeros((ACC - N,), jnp.float32)
    invdeg_ref[...] = jnp.concatenate([rsq * rsq, pad])
    rsq_ref[...] = jnp.concatenate([rsq, pad])
    g0_ref[...] = feat_ref[...] * rsq[:, None]


def _tc_prep(degp, feature):
    return pl.pallas_call(
        _prep_kernel,
        out_shape=(
            jax.ShapeDtypeStruct((N, F), jnp.float32),
            jax.ShapeDtypeStruct((ACC,), jnp.float32),
            jax.ShapeDtypeStruct((ACC,), jnp.float32),
        ),
    )(degp, feature)


def _combine_kernel(p_ref, invdeg_ref, rsq_ref, g_ref, h_ref):
    acc = p_ref[0, :N, :] + p_ref[1, :N, :]
    g_ref[...] = acc * invdeg_ref[:N][:, None]
    h_ref[...] = acc * rsq_ref[:N][:, None]


def _tc_combine(p, invdeg, rsq):
    return pl.pallas_call(
        _combine_kernel,
        out_shape=(
            jax.ShapeDtypeStruct((N, F), jnp.float32),
            jax.ShapeDtypeStruct((N, F), jnp.float32),
        ),
    )(p, invdeg, rsq)


BN = 1000
NB = N // BN


def _enc1_kernel(b1_ref, w1_ref, *refs):
    x_refs = refs[:K + 1]
    h1_ref, ps_ref, psq_ref = refs[K + 1:]
    acc = jnp.broadcast_to(b1_ref[...], (BN, HID)).astype(jnp.float32)
    for k in range(K + 1):
        acc = acc + jnp.dot(x_refs[k][...], w1_ref[k],
                            preferred_element_type=jnp.float32)
    h1 = jnp.where(acc >= 0, acc, SLOPE * acc)
    h1_ref[...] = h1
    ps_ref[...] = jnp.pad(jnp.sum(h1, axis=0, keepdims=True), ((0, 7), (0, 0)))
    psq_ref[...] = jnp.pad(jnp.sum(h1 * h1, axis=0, keepdims=True), ((0, 7), (0, 0)))


def _tc_enc1(xs, w1r, b1):
    x_specs = [pl.BlockSpec((BN, F), lambda i: (i, 0)) for _ in range(K + 1)]
    return pl.pallas_call(
        _enc1_kernel,
        grid=(NB,),
        in_specs=[
            pl.BlockSpec((HID,), lambda i: (0,)),
            pl.BlockSpec((K + 1, F, HID), lambda i: (0, 0, 0)),
        ] + x_specs,
        out_specs=(
            pl.BlockSpec((BN, HID), lambda i: (i, 0)),
            pl.BlockSpec((8, HID), lambda i: (i, 0)),
            pl.BlockSpec((8, HID), lambda i: (i, 0)),
        ),
        out_shape=(
            jax.ShapeDtypeStruct((N, HID), jnp.float32),
            jax.ShapeDtypeStruct((NB * 8, HID), jnp.float32),
            jax.ShapeDtypeStruct((NB * 8, HID), jnp.float32),
        ),
    )(b1, w1r, *xs)


def _enc2_kernel(h1_ref, ps_ref, psq_ref, gamma_ref, beta_ref, w2_ref, b2_ref,
                 wd_ref, bd_ref, zn_ref, r_ref):
    mu = jnp.sum(ps_ref[...], axis=0) / N
    ex2 = jnp.sum(psq_ref[...], axis=0) / N
    var = ex2 - mu * mu
    rstd = lax.rsqrt(var + 1e-5)
    h1 = (h1_ref[...] - mu) * (rstd * gamma_ref[...]) + beta_ref[...]
    z = jnp.dot(h1, w2_ref[...], preferred_element_type=jnp.float32) + b2_ref[...]
    r_ref[...] = jnp.dot(z, wd_ref[...], preferred_element_type=jnp.float32) + bd_ref[...]
    nrm = jnp.sqrt(jnp.sum(z * z, axis=1, keepdims=True))
    zn_ref[...] = z / jnp.maximum(nrm, 1e-12)


def _tc_enc2(h1, ps, psq, gamma, beta, W2, b2, Wd, bd):
    return pl.pallas_call(
        _enc2_kernel,
        grid=(NB,),
        in_specs=[
            pl.BlockSpec((BN, HID), lambda i: (i, 0)),
            pl.BlockSpec((NB * 8, HID), lambda i: (0, 0)),
            pl.BlockSpec((NB * 8, HID), lambda i: (0, 0)),
            pl.BlockSpec((HID,), lambda i: (0,)),
            pl.BlockSpec((HID,), lambda i: (0,)),
            pl.BlockSpec((HID, OUT), lambda i: (0, 0)),
            pl.BlockSpec((OUT,), lambda i: (0,)),
            pl.BlockSpec((OUT, F), lambda i: (0, 0)),
            pl.BlockSpec((F,), lambda i: (0,)),
        ],
        out_specs=(
            pl.BlockSpec((BN, OUT), lambda i: (i, 0)),
            pl.BlockSpec((BN, F), lambda i: (i, 0)),
        ),
        out_shape=(
            jax.ShapeDtypeStruct((N, OUT), jnp.float32),
            jax.ShapeDtypeStruct((N, F), jnp.float32),
        ),
    )(h1, ps, psq, gamma, beta, W2, b2, Wd, bd)


def kernel(feature, edge_index, W1, b1, gamma, beta, W2, b2, Wd, bd):
    row = edge_index[0]
    col = edge_index[1]
    self_mask = row == col
    loop_idx = jnp.arange(N, dtype=jnp.int32)
    trash = N + jnp.mod(row, ACC - N)

    prop_row = jnp.concatenate([row, loop_idx])
    prop_col = jnp.concatenate([jnp.where(self_mask, trash, col), loop_idx])
    deg_row = jnp.concatenate([jnp.where(self_mask, trash, row), loop_idx])

    pad = MPAD - M
    pad_trash = N + jnp.mod(jnp.arange(pad, dtype=jnp.int32), ACC - N)
    pad_src = jnp.mod(jnp.arange(pad, dtype=jnp.int32) * 79, N)
    prop_row = jnp.concatenate([prop_row, pad_src]).reshape(NW, CPT, CHUNK)
    prop_col = jnp.concatenate([prop_col, pad_trash]).reshape(NW, CPT, CHUNK)
    deg_row = jnp.concatenate([deg_row, pad_trash]).reshape(NW, CPT, CHUNK)

    zeros = jnp.zeros((ACC, F), jnp.float32)
    zeros16 = jnp.zeros((ACC, DEGW), jnp.float32)
    ones16 = jnp.ones((CHUNK, DEGW), jnp.float32)

    degp = _sc_degree(deg_row, zeros16, ones16)
    g0, invdeg, rsq = _tc_prep(degp, feature)

    xs = [feature]
    g = g0
    for _ in range(K):
        p = _sc_prop(g, prop_row, prop_col, zeros)
        g, h = _tc_combine(p, invdeg, rsq)
        xs.append(h)

    w1r = W1.reshape(K + 1, F, HID)
    h1, ps, psq = _tc_enc1(xs, w1r, b1)
    zn, r = _tc_enc2(h1, ps, psq, gamma, beta, W2, b2, Wd, bd)
    return (zn, r)
